# Initial kernel scaffold; baseline (speedup 1.0000x reference)
import jax
import jax.numpy as jnp
from jax.experimental import pallas as pl


def kernel(inputs, degs):
    raise NotImplementedError("write your pallas kernel here")


# speedup vs baseline: 1.5788x; 1.5788x over previous
import functools

import jax
import jax.numpy as jnp
from jax import lax
from jax.experimental import pallas as pl
from jax.experimental.pallas import tpu as pltpu
from jax.experimental.pallas import tpu_sc as plsc

NC, NS, L = 2, 16, 16
NW = NC * NS
ROWS, COLS = 16384, 360
RPW = ROWS // NW
CH = 64
NCHUNK = RPW // CH
GROUPS = CH // L
UNROLL = 15

_mesh = plsc.VectorSubcoreMesh(core_axis_name="c", subcore_axis_name="s")


@functools.partial(
    pl.kernel,
    mesh=_mesh,
    compiler_params=pltpu.CompilerParams(needs_layout_passes=False),
    out_type=jax.ShapeDtypeStruct((ROWS,), jnp.float32),
    scratch_types=[
        pltpu.VMEM((CH * COLS,), jnp.float32),
        pltpu.VMEM((CH * COLS,), jnp.float32),
        pltpu.VMEM((COLS,), jnp.int32),
        pltpu.VMEM((RPW,), jnp.float32),
        pltpu.SemaphoreType.DMA,
        pltpu.SemaphoreType.DMA,
    ],
)
def _argmax_deg_kernel(in_hbm, degs_hbm, out_hbm, buf0, buf1, degs_v, out_v,
                       sem0, sem1):
    wid = lax.axis_index("s") * NC + lax.axis_index("c")
    base_row = wid * RPW

    pltpu.sync_copy(degs_hbm, degs_v)

    iota = lax.iota(jnp.int32, L)
    i360 = iota * COLS
    neg_inf = jnp.full((L,), -jnp.inf, jnp.float32)

    bufs = [buf0, buf1]
    sems = [sem0, sem1]
    copies = [None, None]

    def start(ci, b):
        src = in_hbm.at[pl.ds((base_row + ci * CH) * COLS, CH * COLS)]
        copies[b] = pltpu.async_copy(src, bufs[b], sems[b])

    start(0, 0)
    for ci in range(NCHUNK):
        b = ci & 1
        if ci + 1 < NCHUNK:
            start(ci + 1, 1 - b)
        copies[b].wait()
        chunk = bufs[b]

        def group_body(g, _):
            rowbase = i360 + g * (L * COLS)

            def macro(m, carry):
                best, bflat, cur = carry
                for j in range(UNROLL):
                    idx = cur + j if j else cur
                    v = plsc.load_gather(chunk, [idx])
                    p = v > best
                    best = jnp.where(p, v, best)
                    bflat = jnp.where(p, idx, bflat)
                return best, bflat, cur + UNROLL

            best, bflat, _ = lax.fori_loop(
                0, COLS // UNROLL, macro, (neg_inf, rowbase, rowbase))
            col = bflat - rowbase
            d = plsc.load_gather(degs_v, [col])
            out_v[pl.ds(ci * CH + g * L, L)] = d.astype(jnp.float32)
            return 0

        lax.fori_loop(0, GROUPS, group_body, 0)

    pltpu.sync_copy(out_v, out_hbm.at[pl.ds(base_row, RPW)])


@jax.jit
def kernel(inputs, degs):
    return _argmax_deg_kernel(inputs.reshape(-1), degs)


# speedup vs baseline: 1.5818x; 1.0019x over previous
import functools

import jax
import jax.numpy as jnp
from jax import lax
from jax.experimental import pallas as pl
from jax.experimental.pallas import tpu as pltpu
from jax.experimental.pallas import tpu_sc as plsc

NC, NS, L = 2, 16, 16
NW = NC * NS
ROWS, COLS = 16384, 360
RPW = ROWS // NW
CH = 64
NCHUNK = RPW // CH
GROUPS = CH // L
NCOL = 23
SSTR = L + 1

_mesh = plsc.VectorSubcoreMesh(core_axis_name="c", subcore_axis_name="s")


@functools.partial(
    pl.kernel,
    mesh=_mesh,
    compiler_params=pltpu.CompilerParams(needs_layout_passes=False),
    out_type=jax.ShapeDtypeStruct((ROWS,), jnp.float32),
    scratch_types=[
        pltpu.VMEM((CH * COLS + 128,), jnp.float32),
        pltpu.VMEM((CH * COLS + 128,), jnp.float32),
        pltpu.VMEM((L * SSTR,), jnp.float32),
        pltpu.VMEM((L * SSTR,), jnp.int32),
        pltpu.VMEM((COLS,), jnp.int32),
        pltpu.VMEM((RPW,), jnp.float32),
        pltpu.SemaphoreType.DMA,
        pltpu.SemaphoreType.DMA,
    ],
)
def _argmax_deg_kernel(in_hbm, degs_hbm, out_hbm, buf0, buf1, vals_v, cols_v,
                       degs_v, out_v, sem0, sem1):
    wid = lax.axis_index("s") * NC + lax.axis_index("c")
    base_row = wid * RPW

    pltpu.sync_copy(degs_hbm, degs_v)

    iota = lax.iota(jnp.int32, L)
    i17 = iota * SSTR
    neg_inf = jnp.full((L,), -jnp.inf, jnp.float32)
    zero = jnp.zeros((L,), jnp.int32)
    tail_keep = iota < (COLS - (NCOL - 1) * L)

    bufs = [buf0, buf1]
    sems = [sem0, sem1]
    copies = [None, None]

    def start(ci, b):
        src = in_hbm.at[pl.ds((base_row + ci * CH) * COLS, CH * COLS)]
        copies[b] = pltpu.async_copy(src, bufs[b].at[pl.ds(0, CH * COLS)],
                                     sems[b])

    start(0, 0)
    for ci in range(NCHUNK):
        b = ci & 1
        if ci + 1 < NCHUNK:
            start(ci + 1, 1 - b)
        copies[b].wait()
        buf = bufs[b]

        def group_body(g, _):
            def row_body(r, _):
                rowoff = (g * L + r) * COLS
                best = neg_inf
                bchunk = zero
                for c in range(NCOL):
                    v = buf[pl.ds(rowoff + c * L, L)]
                    if c == NCOL - 1:
                        v = jnp.where(tail_keep, v, neg_inf)
                    p = v > best
                    best = jnp.where(p, v, best)
                    bchunk = jnp.where(p, jnp.full((L,), c, jnp.int32), bchunk)
                vals_v[pl.ds(r * SSTR, L)] = best
                cols_v[pl.ds(r * SSTR, L)] = bchunk * L + iota
                return 0

            lax.fori_loop(0, L, row_body, 0)

            best = neg_inf
            bcol = zero
            for j in range(L):
                v = plsc.load_gather(vals_v, [i17 + j if j else i17])
                cj = plsc.load_gather(cols_v, [i17 + j if j else i17])
                pg = v > best
                pe = (v == best) & (cj < bcol)
                p = pg | pe
                best = jnp.where(p, v, best)
                bcol = jnp.where(p, cj, bcol)
            d = plsc.load_gather(degs_v, [bcol])
            out_v[pl.ds(ci * CH + g * L, L)] = d.astype(jnp.float32)
            return 0

        lax.fori_loop(0, GROUPS, group_body, 0)

    pltpu.sync_copy(out_v, out_hbm.at[pl.ds(base_row, RPW)])


@jax.jit
def kernel(inputs, degs):
    return _argmax_deg_kernel(inputs.reshape(-1), degs)


# speedup vs baseline: 2.3678x; 1.4969x over previous
import functools

import jax
import jax.numpy as jnp
from jax import lax
from jax.experimental import pallas as pl
from jax.experimental.pallas import tpu as pltpu
from jax.experimental.pallas import tpu_sc as plsc

NC, NS, L = 2, 16, 16
NW = NC * NS
ROWS, COLS = 16384, 360
RPW = ROWS // NW
CH = 64
NCHUNK = RPW // CH
GROUPS = CH // L
NCOL = 23
TOFF = COLS - L
SSTR = L + 1

_mesh = plsc.VectorSubcoreMesh(core_axis_name="c", subcore_axis_name="s")


@functools.partial(
    pl.kernel,
    mesh=_mesh,
    compiler_params=pltpu.CompilerParams(needs_layout_passes=False),
    out_type=jax.ShapeDtypeStruct((ROWS,), jnp.float32),
    scratch_types=[
        pltpu.VMEM((CH, COLS), jnp.float32),
        pltpu.VMEM((CH, COLS), jnp.float32),
        pltpu.VMEM((L * SSTR,), jnp.float32),
        pltpu.VMEM((L * SSTR,), jnp.int32),
        pltpu.VMEM((COLS,), jnp.int32),
        pltpu.VMEM((RPW,), jnp.float32),
        pltpu.SemaphoreType.DMA,
        pltpu.SemaphoreType.DMA,
    ],
)
def _argmax_deg_kernel(in_hbm, degs_hbm, out_hbm, buf0, buf1, vals_v, cols_v,
                       degs_v, out_v, sem0, sem1):
    wid = lax.axis_index("s") * NC + lax.axis_index("c")
    base_row = wid * RPW

    pltpu.sync_copy(degs_hbm, degs_v)

    iota = lax.iota(jnp.int32, L)
    i17 = iota * SSTR
    neg_inf = jnp.full((L,), -jnp.inf, jnp.float32)
    zero = jnp.zeros((L,), jnp.int32)

    bufs = [buf0, buf1]
    sems = [sem0, sem1]
    copies = [None, None]

    def start(ci, b):
        src = in_hbm.at[pl.ds(base_row + ci * CH, CH)]
        copies[b] = pltpu.async_copy(src, bufs[b], sems[b])

    start(0, 0)
    for ci in range(NCHUNK):
        b = ci & 1
        if ci + 1 < NCHUNK:
            start(ci + 1, 1 - b)
        copies[b].wait()
        buf = bufs[b]

        def group_body(g, _):
            def row_body(r, _):
                row = g * L + r
                best = neg_inf
                bbase = zero
                for c in range(NCOL):
                    off = c * L if c < NCOL - 1 else TOFF
                    v = buf[row, pl.ds(off, L)]
                    p = v > best
                    best = jnp.where(p, v, best)
                    bbase = jnp.where(p, jnp.full((L,), off, jnp.int32), bbase)
                vals_v[pl.ds(r * SSTR, L)] = best
                cols_v[pl.ds(r * SSTR, L)] = bbase + iota
                return 0

            lax.fori_loop(0, L, row_body, 0)

            best = neg_inf
            bcol = zero
            for j in range(L):
                v = plsc.load_gather(vals_v, [i17 + j if j else i17])
                cj = plsc.load_gather(cols_v, [i17 + j if j else i17])
                pg = v > best
                pe = (v == best) & (cj < bcol)
                p = pg | pe
                best = jnp.where(p, v, best)
                bcol = jnp.where(p, cj, bcol)
            d = plsc.load_gather(degs_v, [bcol])
            out_v[pl.ds(ci * CH + g * L, L)] = d.astype(jnp.float32)
            return 0

        lax.fori_loop(0, GROUPS, group_body, 0)

    pltpu.sync_copy(out_v, out_hbm.at[pl.ds(base_row, RPW)])


@jax.jit
def kernel(inputs, degs):
    return _argmax_deg_kernel(inputs, degs)


# speedup vs baseline: 2.3735x; 1.0024x over previous
import functools

import jax
import jax.numpy as jnp
from jax import lax
from jax.experimental import pallas as pl
from jax.experimental.pallas import tpu as pltpu
from jax.experimental.pallas import tpu_sc as plsc

NC, NS, L = 2, 16, 16
NW = NC * NS
ROWS, COLS = 16384, 360
RPW = ROWS // NW
CH = 64
NCHUNK = RPW // CH
GROUPS = CH // L
NCOL = 23
TOFF = COLS - L
SSTR = L + 1

_mesh = plsc.VectorSubcoreMesh(core_axis_name="c", subcore_axis_name="s")


@functools.partial(
    pl.kernel,
    mesh=_mesh,
    compiler_params=pltpu.CompilerParams(needs_layout_passes=False,
                                         use_tc_tiling_on_sc=True),
    out_type=jax.ShapeDtypeStruct((ROWS,), jnp.float32),
    scratch_types=[
        pltpu.VMEM((CH, COLS), jnp.float32),
        pltpu.VMEM((CH, COLS), jnp.float32),
        pltpu.VMEM((L * SSTR,), jnp.float32),
        pltpu.VMEM((L * SSTR,), jnp.int32),
        pltpu.VMEM((COLS,), jnp.int32),
        pltpu.VMEM((RPW,), jnp.float32),
        pltpu.SemaphoreType.DMA,
        pltpu.SemaphoreType.DMA,
    ],
)
def _argmax_deg_kernel(in_hbm, degs_hbm, out_hbm, buf0, buf1, vals_v, cols_v,
                       degs_v, out_v, sem0, sem1):
    wid = lax.axis_index("s") * NC + lax.axis_index("c")
    base_row = wid * RPW

    pltpu.sync_copy(degs_hbm, degs_v)

    iota = lax.iota(jnp.int32, L)
    i17 = iota * SSTR
    neg_inf = jnp.full((L,), -jnp.inf, jnp.float32)
    zero = jnp.zeros((L,), jnp.int32)

    bufs = [buf0, buf1]
    sems = [sem0, sem1]
    copies = [None, None]

    def start(ci, b):
        src = in_hbm.at[pl.ds(base_row + ci * CH, CH)]
        copies[b] = pltpu.async_copy(src, bufs[b], sems[b])

    start(0, 0)
    for ci in range(NCHUNK):
        b = ci & 1
        if ci + 1 < NCHUNK:
            start(ci + 1, 1 - b)
        copies[b].wait()
        buf = bufs[b]

        def group_body(g, _):
            def row_body(r, _):
                row = g * L + r
                best = neg_inf
                bbase = zero
                for c in range(NCOL):
                    off = c * L if c < NCOL - 1 else TOFF
                    v = buf[row, pl.ds(off, L)]
                    p = v > best
                    best = jnp.where(p, v, best)
                    bbase = jnp.where(p, jnp.full((L,), off, jnp.int32), bbase)
                vals_v[pl.ds(r * SSTR, L)] = best
                cols_v[pl.ds(r * SSTR, L)] = bbase + iota
                return 0

            lax.fori_loop(0, L, row_body, 0)

            best = neg_inf
            bcol = zero
            for j in range(L):
                v = plsc.load_gather(vals_v, [i17 + j if j else i17])
                cj = plsc.load_gather(cols_v, [i17 + j if j else i17])
                pg = v > best
                pe = (v == best) & (cj < bcol)
                p = pg | pe
                best = jnp.where(p, v, best)
                bcol = jnp.where(p, cj, bcol)
            d = plsc.load_gather(degs_v, [bcol])
            out_v[pl.ds(ci * CH + g * L, L)] = d.astype(jnp.float32)
            return 0

        lax.fori_loop(0, GROUPS, group_body, 0)

    pltpu.sync_copy(out_v, out_hbm.at[pl.ds(base_row, RPW)])


@jax.jit
def kernel(inputs, degs):
    return _argmax_deg_kernel(inputs, degs)


# speedup vs baseline: 2.6341x; 1.1098x over previous
import functools

import jax
import jax.numpy as jnp
from jax import lax
from jax.experimental import pallas as pl
from jax.experimental.pallas import tpu as pltpu
from jax.experimental.pallas import tpu_sc as plsc

NC, NS, L = 2, 16, 16
NW = NC * NS
ROWS, COLS = 16384, 360
RPW = ROWS // NW
CH = 64
NCHUNK = RPW // CH
GROUPS = CH // L
NCOL = 23
TOFF = COLS - L
SSTR = L + 1

_mesh = plsc.VectorSubcoreMesh(core_axis_name="c", subcore_axis_name="s")


@functools.partial(
    pl.kernel,
    mesh=_mesh,
    compiler_params=pltpu.CompilerParams(needs_layout_passes=False,
                                         use_tc_tiling_on_sc=True),
    out_type=jax.ShapeDtypeStruct((ROWS,), jnp.float32),
    scratch_types=[
        pltpu.VMEM((CH, COLS), jnp.float32),
        pltpu.VMEM((CH, COLS), jnp.float32),
        pltpu.VMEM((L * SSTR,), jnp.float32),
        pltpu.VMEM((L * SSTR,), jnp.int32),
        pltpu.VMEM((COLS,), jnp.int32),
        pltpu.VMEM((RPW,), jnp.float32),
        pltpu.SemaphoreType.DMA,
        pltpu.SemaphoreType.DMA,
    ],
)
def _argmax_deg_kernel(in_hbm, degs_hbm, out_hbm, buf0, buf1, vals_v, cols_v,
                       degs_v, out_v, sem0, sem1):
    wid = lax.axis_index("s") * NC + lax.axis_index("c")
    base_row = wid * RPW

    pltpu.sync_copy(degs_hbm, degs_v)

    iota = lax.iota(jnp.int32, L)
    i17 = iota * SSTR
    neg_inf = jnp.full((L,), -jnp.inf, jnp.float32)
    zero = jnp.zeros((L,), jnp.int32)

    bufs = [buf0, buf1]
    sems = [sem0, sem1]
    copies = [None, None]

    def start(ci, b):
        src = in_hbm.at[pl.ds(base_row + ci * CH, CH)]
        copies[b] = pltpu.async_copy(src, bufs[b], sems[b])

    start(0, 0)
    for ci in range(NCHUNK):
        b = ci & 1
        if ci + 1 < NCHUNK:
            start(ci + 1, 1 - b)
        copies[b].wait()
        buf = bufs[b]

        def group_body(g, _):
            def rowpair_body(r2, _):
                r0 = 2 * r2
                row0 = g * L + r0
                best0 = neg_inf
                best1 = neg_inf
                bbase0 = zero
                bbase1 = zero
                for c in range(NCOL):
                    off = c * L if c < NCOL - 1 else TOFF
                    offv = jnp.full((L,), off, jnp.int32)
                    v0 = buf[row0, pl.ds(off, L)]
                    v1 = buf[row0 + 1, pl.ds(off, L)]
                    p0 = v0 > best0
                    p1 = v1 > best1
                    best0 = jnp.where(p0, v0, best0)
                    best1 = jnp.where(p1, v1, best1)
                    bbase0 = jnp.where(p0, offv, bbase0)
                    bbase1 = jnp.where(p1, offv, bbase1)
                vals_v[pl.ds(r0 * SSTR, L)] = best0
                cols_v[pl.ds(r0 * SSTR, L)] = bbase0 + iota
                vals_v[pl.ds((r0 + 1) * SSTR, L)] = best1
                cols_v[pl.ds((r0 + 1) * SSTR, L)] = bbase1 + iota
                return 0

            lax.fori_loop(0, L // 2, rowpair_body, 0)

            best = neg_inf
            bcol = zero
            for j in range(L):
                v = plsc.load_gather(vals_v, [i17 + j if j else i17])
                cj = plsc.load_gather(cols_v, [i17 + j if j else i17])
                pg = v > best
                pe = (v == best) & (cj < bcol)
                p = pg | pe
                best = jnp.where(p, v, best)
                bcol = jnp.where(p, cj, bcol)
            d = plsc.load_gather(degs_v, [bcol])
            out_v[pl.ds(ci * CH + g * L, L)] = d.astype(jnp.float32)
            return 0

        lax.fori_loop(0, GROUPS, group_body, 0)

    pltpu.sync_copy(out_v, out_hbm.at[pl.ds(base_row, RPW)])


@jax.jit
def kernel(inputs, degs):
    return _argmax_deg_kernel(inputs, degs)


# speedup vs baseline: 2.6393x; 1.0020x over previous
import functools

import jax
import jax.numpy as jnp
from jax import lax
from jax.experimental import pallas as pl
from jax.experimental.pallas import tpu as pltpu
from jax.experimental.pallas import tpu_sc as plsc

NC, NS, L = 2, 16, 16
NW = NC * NS
ROWS, COLS = 16384, 360
RPW = ROWS // NW
CH = 64
NCHUNK = RPW // CH
GROUPS = CH // L
NCOL = 23
TOFF = COLS - L
SSTR = L + 1

_mesh = plsc.VectorSubcoreMesh(core_axis_name="c", subcore_axis_name="s")


@functools.partial(
    pl.kernel,
    mesh=_mesh,
    compiler_params=pltpu.CompilerParams(needs_layout_passes=False,
                                         use_tc_tiling_on_sc=True),
    out_type=jax.ShapeDtypeStruct((ROWS,), jnp.float32),
    scratch_types=[
        pltpu.VMEM((CH, COLS), jnp.float32),
        pltpu.VMEM((CH, COLS), jnp.float32),
        pltpu.VMEM((L * SSTR,), jnp.float32),
        pltpu.VMEM((L * SSTR,), jnp.int32),
        pltpu.VMEM((COLS,), jnp.int32),
        pltpu.VMEM((RPW,), jnp.float32),
        pltpu.SemaphoreType.DMA,
        pltpu.SemaphoreType.DMA,
    ],
)
def _argmax_deg_kernel(in_hbm, degs_hbm, out_hbm, buf0, buf1, vals_v, cols_v,
                       degs_v, out_v, sem0, sem1):
    wid = lax.axis_index("s") * NC + lax.axis_index("c")
    base_row = wid * RPW

    pltpu.sync_copy(degs_hbm, degs_v)

    iota = lax.iota(jnp.int32, L)
    i17 = iota * SSTR
    neg_inf = jnp.full((L,), -jnp.inf, jnp.float32)
    zero = jnp.zeros((L,), jnp.int32)

    bufs = [buf0, buf1]
    sems = [sem0, sem1]
    copies = [None, None]

    def start(ci, b):
        src = in_hbm.at[pl.ds(base_row + ci * CH, CH)]
        copies[b] = pltpu.async_copy(src, bufs[b], sems[b])

    start(0, 0)
    for ci in range(NCHUNK):
        b = ci & 1
        if ci + 1 < NCHUNK:
            start(ci + 1, 1 - b)
        copies[b].wait()
        buf = bufs[b]

        def group_body(g, _):
            def rowquad_body(r4, _):
                r0 = 4 * r4
                row0 = g * L + r0
                best = [neg_inf] * 4
                bbase = [zero] * 4
                for c in range(NCOL):
                    off = c * L if c < NCOL - 1 else TOFF
                    offv = jnp.full((L,), off, jnp.int32)
                    v = [buf[row0 + k, pl.ds(off, L)] for k in range(4)]
                    p = [v[k] > best[k] for k in range(4)]
                    best = [jnp.where(p[k], v[k], best[k]) for k in range(4)]
                    bbase = [jnp.where(p[k], offv, bbase[k])
                             for k in range(4)]
                for k in range(4):
                    vals_v[pl.ds((r0 + k) * SSTR, L)] = best[k]
                    cols_v[pl.ds((r0 + k) * SSTR, L)] = bbase[k] + iota
                return 0

            lax.fori_loop(0, L // 4, rowquad_body, 0)

            best = neg_inf
            bcol = zero
            for j in range(L):
                v = plsc.load_gather(vals_v, [i17 + j if j else i17])
                cj = plsc.load_gather(cols_v, [i17 + j if j else i17])
                pg = v > best
                pe = (v == best) & (cj < bcol)
                p = pg | pe
                best = jnp.where(p, v, best)
                bcol = jnp.where(p, cj, bcol)
            d = plsc.load_gather(degs_v, [bcol])
            out_v[pl.ds(ci * CH + g * L, L)] = d.astype(jnp.float32)
            return 0

        lax.fori_loop(0, GROUPS, group_body, 0)

    pltpu.sync_copy(out_v, out_hbm.at[pl.ds(base_row, RPW)])


@jax.jit
def kernel(inputs, degs):
    return _argmax_deg_kernel(inputs, degs)
